# Initial kernel scaffold; baseline (speedup 1.0000x reference)
import jax
import jax.numpy as jnp
from jax import lax
from jax.experimental import pallas as pl
from jax.experimental.pallas import tpu as pltpu
from jax.experimental.pallas import tpu_sc as plsc

f32 = jnp.float32
i32 = jnp.int32

NU = 50000
NI = 50000
NN = NU + NI
D = 32
H = 16
NLAYERS = 3
E = 1600000
B = 4096

NC = 2
NS = 16

SUB = 128
NSTREAM = 16
C = SUB * NSTREAM
CHUNKS = 50
EPAD = NS * CHUNKS * C
ROWS_PT = NN // NS

_mesh = plsc.VectorSubcoreMesh(
    core_axis_name="c", subcore_axis_name="s", num_cores=NC, num_subcores=NS
)


def _layer_body(h0, h1, col3, row3, val2, o0, o1,
                col_v, row_v, val_v, rows_v, acc, sem):
    c = lax.axis_index("c")
    s = lax.axis_index("s")

    @pl.loop(0, C, step=16)
    def _zero(j0):
        eidx = j0 + lax.iota(i32, 16)
        zero = jnp.zeros((16,), f32)
        for k in range(H):
            plsc.store_scatter(rows_v, [eidx, jnp.full((16,), k, i32)], zero)

    zb = s * ROWS_PT
    pltpu.sync_copy(rows_v, acc.at[pl.ds(zb, C)])
    pltpu.sync_copy(rows_v, acc.at[pl.ds(zb + C, C)])
    pltpu.sync_copy(rows_v, acc.at[pl.ds(zb + 2 * C, C)])
    pltpu.sync_copy(rows_v.at[pl.ds(0, ROWS_PT - 3 * C)],
                    acc.at[pl.ds(zb + 3 * C, ROWS_PT - 3 * C)])
    plsc.subcore_barrier()

    @pl.loop(0, CHUNKS)
    def _chunk(ch):
        pltpu.sync_copy(col3.at[s, ch], col_v)
        pltpu.sync_copy(row3.at[s, ch], row_v)
        pltpu.sync_copy(val2.at[s, ch], val_v)

        def _gather_from(tab):
            descs = [
                pltpu.async_copy(tab.at[col_v.at[j]],
                                 rows_v.at[pl.ds(j * SUB, SUB)], sem)
                for j in range(NSTREAM)
            ]
            for d in descs:
                d.wait()

        @pl.when(c == 0)
        def _g0():
            _gather_from(h0)

        @pl.when(c == 1)
        def _g1():
            _gather_from(h1)

        @pl.loop(0, C, step=16)
        def _scale(j0):
            eidx = j0 + lax.iota(i32, 16)
            v = val_v[pl.ds(j0, 16)]
            for k in range(H):
                kidx = jnp.full((16,), k, i32)
                r = plsc.load_gather(rows_v, [eidx, kidx])
                plsc.store_scatter(rows_v, [eidx, kidx], r * v)

        descs = [
            pltpu.async_copy(rows_v.at[pl.ds(j * SUB, SUB)],
                             acc.at[row_v.at[j]], sem, add=True)
            for j in range(NSTREAM)
        ]
        for d in descs:
            d.wait()

    plsc.subcore_barrier()

    @pl.when(c == 0)
    def _w0():
        pltpu.sync_copy(acc.at[pl.ds(s * ROWS_PT, ROWS_PT)],
                        o0.at[pl.ds(s * ROWS_PT, ROWS_PT)])

    @pl.when(c == 1)
    def _w1():
        pltpu.sync_copy(acc.at[pl.ds(s * ROWS_PT, ROWS_PT)],
                        o1.at[pl.ds(s * ROWS_PT, ROWS_PT)])


_layer = pl.kernel(
    _layer_body,
    out_type=(
        jax.ShapeDtypeStruct((NN, H), f32),
        jax.ShapeDtypeStruct((NN, H), f32),
    ),
    mesh=_mesh,
    scratch_types=[
        pltpu.VMEM((NSTREAM, SUB), i32),
        pltpu.VMEM((NSTREAM, SUB), i32),
        pltpu.VMEM((C,), f32),
        pltpu.VMEM((C, H), f32),
        pltpu.VMEM_SHARED((NN, H), f32),
        pltpu.SemaphoreType.DMA,
    ],
)

BPW = B // (NC * NS)


def _score_body(e00, e01, e10, e11, e20, e21, e30, e31, uid2, iid2, out,
                idx_u, idx_i, gbuf, sc_v, sem):
    c = lax.axis_index("c")
    s = lax.axis_index("s")
    w = s * NC + c

    pltpu.sync_copy(uid2.at[w], idx_u)
    pltpu.sync_copy(iid2.at[w], idx_i)

    tabs = [e00, e01, e10, e11, e20, e21, e30, e31]
    descs = []
    for t in range(8):
        descs.append(pltpu.async_copy(tabs[t].at[idx_u], gbuf.at[t], sem))
        descs.append(pltpu.async_copy(tabs[t].at[idx_i], gbuf.at[8 + t], sem))
    for d in descs:
        d.wait()

    @pl.loop(0, BPW, step=16)
    def _dot(j0):
        eidx = j0 + lax.iota(i32, 16)
        tot = jnp.zeros((16,), f32)
        for k in range(H):
            kidx = jnp.full((16,), k, i32)

            def _lsum(base):
                r = plsc.load_gather(gbuf, [jnp.full((16,), base, i32), eidx, kidx])
                for l in range(1, 4):
                    r = r + plsc.load_gather(
                        gbuf, [jnp.full((16,), base + 2 * l, i32), eidx, kidx])
                return r

            u0 = _lsum(0)
            u1 = _lsum(1)
            i0 = _lsum(8)
            i1 = _lsum(9)
            tot = tot + u0 * i0 + u1 * i1
        sc_v[pl.ds(j0, 16)] = tot * (1.0 / 16.0)

    pltpu.sync_copy(sc_v, out.at[pl.ds(w * BPW, BPW)])


_score = pl.kernel(
    _score_body,
    out_type=jax.ShapeDtypeStruct((B,), f32),
    mesh=_mesh,
    scratch_types=[
        pltpu.VMEM((BPW,), i32),
        pltpu.VMEM((BPW,), i32),
        pltpu.VMEM((16, BPW, H), f32),
        pltpu.VMEM((BPW,), f32),
        pltpu.SemaphoreType.DMA,
    ],
)


def kernel(user_ids, item_ids, adj_indices, adj_values, user_emb_w, item_emb_w):
    row = adj_indices[0].astype(i32)
    col = adj_indices[1].astype(i32)
    pad = EPAD - E
    colp = jnp.concatenate([col, jnp.zeros((pad,), i32)])
    rowp = jnp.concatenate([row, jnp.zeros((pad,), i32)])
    valp = jnp.concatenate([adj_values.astype(f32), jnp.zeros((pad,), f32)])
    col3 = colp.reshape(NS, CHUNKS, NSTREAM, SUB)
    row3 = rowp.reshape(NS, CHUNKS, NSTREAM, SUB)
    val2 = valp.reshape(NS, CHUNKS, C)

    a0 = jnp.concatenate([user_emb_w[:, :H], item_emb_w[:, :H]], axis=0)
    a1 = jnp.concatenate([user_emb_w[:, H:], item_emb_w[:, H:]], axis=0)
    embs = [(a0, a1)]
    for _ in range(NLAYERS):
        a0, a1 = _layer(a0, a1, col3, row3, val2)
        embs.append((a0, a1))

    uid2 = user_ids.astype(i32).reshape(NC * NS, BPW)
    iid2 = (item_ids.astype(i32) + NU).reshape(NC * NS, BPW)
    return _score(embs[0][0], embs[0][1], embs[1][0], embs[1][1],
                  embs[2][0], embs[2][1], embs[3][0], embs[3][1],
                  uid2, iid2)


# speedup vs baseline: 3.5307x; 3.5307x over previous
import jax
import jax.numpy as jnp
from jax import lax
from jax.experimental import pallas as pl
from jax.experimental.pallas import tpu as pltpu
from jax.experimental.pallas import tpu_sc as plsc

f32 = jnp.float32
i32 = jnp.int32

NU = 50000
NI = 50000
NN = NU + NI
D = 32
H = 16
NLAYERS = 3
E = 1600000
B = 4096

NC = 2
NS = 16

SUB = 128
NSTREAM = 16
C = SUB * NSTREAM
CHUNKS = 50
EPAD = NS * CHUNKS * C

NH = NN // 2
ACC_ROWS = 50016
ZPT = 3128
ZPT_LAST = ACC_ROWS - (NS - 1) * ZPT
WPT = 3128
WPT_LAST = NH - (NS - 1) * WPT

_mesh = plsc.VectorSubcoreMesh(
    core_axis_name="c", subcore_axis_name="s", num_cores=NC, num_subcores=NS
)

_BCAST_DN = lax.GatherDimensionNumbers(
    offset_dims=(), collapsed_slice_dims=(0,), start_index_map=(0,))


def _bcast_lane(v, lane):
    idx = jnp.full((16, 1), lane, i32)
    return lax.gather(v, idx, _BCAST_DN, (1,),
                      mode=lax.GatherScatterMode.PROMISE_IN_BOUNDS)


def _layer_body(h0, h1, col3, row3, val2, o0, o1,
                col_v, row_v, val_v, rows_v, acc, sem):
    c = lax.axis_index("c")
    s = lax.axis_index("s")

    for p in range(2):
        @pl.loop(0, C)
        def _zero(j):
            rows_v[j, :] = jnp.zeros((16,), f32)

        zb = s * ZPT
        pltpu.sync_copy(rows_v, acc.at[pl.ds(zb, C)])

        @pl.when(s < NS - 1)
        def _zr1():
            pltpu.sync_copy(rows_v.at[pl.ds(0, ZPT - C)],
                            acc.at[pl.ds(zb + C, ZPT - C)])

        @pl.when(s == NS - 1)
        def _zr2():
            pltpu.sync_copy(rows_v.at[pl.ds(0, ZPT_LAST - C)],
                            acc.at[pl.ds(zb + C, ZPT_LAST - C)])

        plsc.subcore_barrier()

        @pl.loop(0, CHUNKS)
        def _chunk(ch):
            pltpu.sync_copy(col3.at[s, ch], col_v)
            pltpu.sync_copy(row3.at[s, ch], row_v)
            pltpu.sync_copy(val2.at[s, ch], val_v)

            for r in range(NSTREAM):
                for k in range(SUB // 16):
                    rv = row_v[r, pl.ds(k * 16, 16)]
                    if p == 0:
                        rl = jnp.where(rv < NH, rv, NH)
                    else:
                        rl = jnp.where(rv < NH, NH, rv - NH)
                    row_v[r, pl.ds(k * 16, 16)] = rl

            def _gather_from(tab):
                descs = [
                    pltpu.async_copy(tab.at[col_v.at[j]],
                                     rows_v.at[pl.ds(j * SUB, SUB)], sem)
                    for j in range(NSTREAM)
                ]
                for d in descs:
                    d.wait()

            @pl.when(c == 0)
            def _g0():
                _gather_from(h0)

            @pl.when(c == 1)
            def _g1():
                _gather_from(h1)

            @pl.loop(0, C, step=16)
            def _scale(j0):
                v = val_v[pl.ds(j0, 16)]
                for jj in range(16):
                    b = _bcast_lane(v, jj)
                    rows_v[j0 + jj, :] = rows_v[j0 + jj, :] * b

            descs = [
                pltpu.async_copy(rows_v.at[pl.ds(j * SUB, SUB)],
                                 acc.at[row_v.at[j]], sem, add=True)
                for j in range(NSTREAM)
            ]
            for d in descs:
                d.wait()

        plsc.subcore_barrier()

        def _writeback(dst):
            base = s * WPT
            pltpu.sync_copy(acc.at[pl.ds(base, C)],
                            dst.at[pl.ds(p * NH + base, C)])

            @pl.when(s < NS - 1)
            def _r1():
                pltpu.sync_copy(acc.at[pl.ds(base + C, WPT - C)],
                                dst.at[pl.ds(p * NH + base + C, WPT - C)])

            @pl.when(s == NS - 1)
            def _r2():
                pltpu.sync_copy(acc.at[pl.ds(base + C, WPT_LAST - C)],
                                dst.at[pl.ds(p * NH + base + C, WPT_LAST - C)])

        @pl.when(c == 0)
        def _w0():
            _writeback(o0)

        @pl.when(c == 1)
        def _w1():
            _writeback(o1)


_layer = pl.kernel(
    _layer_body,
    out_type=(
        jax.ShapeDtypeStruct((NN, H), f32),
        jax.ShapeDtypeStruct((NN, H), f32),
    ),
    mesh=_mesh,
    compiler_params=pltpu.CompilerParams(use_tc_tiling_on_sc=False, needs_layout_passes=False),
    scratch_types=[
        pltpu.VMEM((NSTREAM, SUB), i32),
        pltpu.VMEM((NSTREAM, SUB), i32),
        pltpu.VMEM((C,), f32),
        pltpu.VMEM((C, H), f32),
        pltpu.VMEM_SHARED((ACC_ROWS, H), f32),
        pltpu.SemaphoreType.DMA,
    ],
)

BPW = B // (NC * NS)


def _score_body(e00, e01, e10, e11, e20, e21, e30, e31, uids, iids, out,
                idx_u, idx_i, gbuf, sc_v, sem):
    c = lax.axis_index("c")
    s = lax.axis_index("s")
    w = s * NC + c

    pltpu.sync_copy(uids.at[pl.ds(w * BPW, BPW)], idx_u)
    pltpu.sync_copy(iids.at[pl.ds(w * BPW, BPW)], idx_i)

    tabs = [e00, e01, e10, e11, e20, e21, e30, e31]
    descs = []
    for t in range(8):
        descs.append(pltpu.async_copy(tabs[t].at[idx_u], gbuf.at[t], sem))
        descs.append(pltpu.async_copy(tabs[t].at[idx_i], gbuf.at[8 + t], sem))
    for d in descs:
        d.wait()

    @pl.loop(0, BPW, step=16)
    def _dot(j0):
        lane = lax.iota(i32, 16)
        tot = jnp.zeros((16,), f32)
        for jj in range(16):
            j = j0 + jj
            u0 = gbuf[0, j, :] + gbuf[2, j, :] + gbuf[4, j, :] + gbuf[6, j, :]
            u1 = gbuf[1, j, :] + gbuf[3, j, :] + gbuf[5, j, :] + gbuf[7, j, :]
            i0 = gbuf[8, j, :] + gbuf[10, j, :] + gbuf[12, j, :] + gbuf[14, j, :]
            i1 = gbuf[9, j, :] + gbuf[11, j, :] + gbuf[13, j, :] + gbuf[15, j, :]
            p = u0 * i0 + u1 * i1
            ssum = jnp.sum(p) * (1.0 / 16.0)
            tot = jnp.where(lane == jj, lax.broadcast_in_dim(ssum, (16,), ()), tot)
        sc_v[pl.ds(j0, 16)] = tot

    pltpu.sync_copy(sc_v, out.at[pl.ds(w * BPW, BPW)])


_score = pl.kernel(
    _score_body,
    out_type=jax.ShapeDtypeStruct((B,), f32),
    mesh=_mesh,
    compiler_params=pltpu.CompilerParams(use_tc_tiling_on_sc=False, needs_layout_passes=False),
    scratch_types=[
        pltpu.VMEM((BPW,), i32),
        pltpu.VMEM((BPW,), i32),
        pltpu.VMEM((16, BPW, H), f32),
        pltpu.VMEM((BPW,), f32),
        pltpu.SemaphoreType.DMA,
    ],
)


def kernel(user_ids, item_ids, adj_indices, adj_values, user_emb_w, item_emb_w):
    row = adj_indices[0].astype(i32)
    col = adj_indices[1].astype(i32)
    pad = EPAD - E
    colp = jnp.concatenate([col, jnp.zeros((pad,), i32)])
    rowp = jnp.concatenate([row, jnp.zeros((pad,), i32)])
    valp = jnp.concatenate([adj_values.astype(f32), jnp.zeros((pad,), f32)])
    col3 = colp.reshape(NS, CHUNKS, NSTREAM, SUB)
    row3 = rowp.reshape(NS, CHUNKS, NSTREAM, SUB)
    val2 = valp.reshape(NS, CHUNKS, C)

    a0 = jnp.concatenate([user_emb_w[:, :H], item_emb_w[:, :H]], axis=0)
    a1 = jnp.concatenate([user_emb_w[:, H:], item_emb_w[:, H:]], axis=0)
    embs = [(a0, a1)]
    for _ in range(NLAYERS):
        a0, a1 = _layer(a0, a1, col3, row3, val2)
        embs.append((a0, a1))

    uids = user_ids.astype(i32)
    iids = item_ids.astype(i32) + NU
    return _score(embs[0][0], embs[0][1], embs[1][0], embs[1][1],
                  embs[2][0], embs[2][1], embs[3][0], embs[3][1],
                  uids, iids)


# speedup vs baseline: 3.9422x; 1.1165x over previous
import jax
import jax.numpy as jnp
from jax import lax
from jax.experimental import pallas as pl
from jax.experimental.pallas import tpu as pltpu
from jax.experimental.pallas import tpu_sc as plsc

f32 = jnp.float32
i32 = jnp.int32

NU = 50000
NI = 50000
NN = NU + NI
D = 32
H = 16
NLAYERS = 3
E = 1600000
B = 4096

NC = 2
NS = 16

SUB = 128
NSTREAM = 16
C = SUB * NSTREAM
CHUNKS = 50
EPAD = NS * CHUNKS * C

NH = NN // 2
ACC_ROWS = 50016
ZPT = 3128
ZPT_LAST = ACC_ROWS - (NS - 1) * ZPT
WPT = 3128
WPT_LAST = NH - (NS - 1) * WPT

_mesh = plsc.VectorSubcoreMesh(
    core_axis_name="c", subcore_axis_name="s", num_cores=NC, num_subcores=NS
)

_BCAST_DN = lax.GatherDimensionNumbers(
    offset_dims=(), collapsed_slice_dims=(0,), start_index_map=(0,))


def _bcast_lane(v, lane):
    idx = jnp.full((16, 1), lane, i32)
    return lax.gather(v, idx, _BCAST_DN, (1,),
                      mode=lax.GatherScatterMode.PROMISE_IN_BOUNDS)


def _layer_body(h0, h1, edata, o0, o1,
                ed0, ed1, rv0, rv1, acc, sg0, sg1, ss0, ss1):
    c = lax.axis_index("c")
    s = lax.axis_index("s")
    edv = [ed0, ed1]
    rv = [rv0, rv1]
    sg = [sg0, sg1]
    ss = [ss0, ss1]

    def _transform(bi, p):
        lo = p * NH

        @pl.loop(0, C, step=16)
        def _t(j0):
            r = j0 >> 7
            kk = j0 & (SUB - 1)
            rvec = edv[bi][1, r, pl.ds(kk, 16)]
            inr = (rvec >= lo) & (rvec < lo + NH)
            edv[bi][1, r, pl.ds(kk, 16)] = jnp.where(inr, rvec - lo, NH)

    def _fire_gathers(bi):
        @pl.when(c == 0)
        def _g0():
            @pl.loop(0, NSTREAM)
            def _g0j(j):
                pltpu.async_copy(h0.at[edv[bi].at[0, j]],
                                 rv[bi].at[pl.ds(j * SUB, SUB)], sg[bi])

        @pl.when(c == 1)
        def _g1():
            @pl.loop(0, NSTREAM)
            def _g1j(j):
                pltpu.async_copy(h1.at[edv[bi].at[0, j]],
                                 rv[bi].at[pl.ds(j * SUB, SUB)], sg[bi])

    def _wait_gathers(bi):
        @pl.loop(0, NSTREAM)
        def _wgj(j):
            pltpu.make_async_copy(h0.at[edv[bi].at[0, j]],
                                  rv[bi].at[pl.ds(j * SUB, SUB)],
                                  sg[bi]).wait()

    def _wait_scatter(bi):
        @pl.loop(0, NSTREAM)
        def _wsj(j):
            pltpu.make_async_copy(rv[bi].at[pl.ds(j * SUB, SUB)],
                                  acc.at[edv[bi].at[1, j]], ss[bi]).wait()

    lane = lax.iota(i32, 16)

    @pl.loop(0, 2)
    def _pass(p):
        @pl.loop(0, C)
        def _zf(j):
            rv1[j, :] = jnp.zeros((16,), f32)

        zb = s * ZPT
        pltpu.sync_copy(rv1, acc.at[pl.ds(zb, C)])

        @pl.when(s < NS - 1)
        def _zr1():
            pltpu.sync_copy(rv1.at[pl.ds(0, ZPT - C)],
                            acc.at[pl.ds(zb + C, ZPT - C)])

        @pl.when(s == NS - 1)
        def _zr2():
            pltpu.sync_copy(rv1.at[pl.ds(0, ZPT_LAST - C)],
                            acc.at[pl.ds(zb + C, ZPT_LAST - C)])

        pltpu.sync_copy(edata.at[s, 0], edv[0])
        _transform(0, p)
        _fire_gathers(0)

        plsc.subcore_barrier()

        @pl.loop(0, CHUNKS, step=2)
        def _chunk2(ch0):
            for b in range(2):
                ch = ch0 + b
                nb = 1 - b

                @pl.when(ch + 1 < CHUNKS)
                def _fire_next():
                    @pl.when(ch >= 1)
                    def _ws():
                        _wait_scatter(nb)

                    pltpu.sync_copy(edata.at[s, ch + 1], edv[nb])
                    _transform(nb, p)
                    _fire_gathers(nb)

                _wait_gathers(b)

                @pl.loop(0, NSTREAM)
                def _scale_scatter(j):
                    @pl.loop(0, SUB, step=16)
                    def _scale(jj):
                        vi = edv[b][2, j, pl.ds(jj, 16)]
                        v = plsc.bitcast(vi, f32)
                        eidx = (j * SUB) + jj + lane
                        for k2 in range(H):
                            kf = jnp.full((16,), k2, i32)
                            g = plsc.load_gather(rv[b], [eidx, kf])
                            plsc.store_scatter(rv[b], [eidx, kf], g * v)

                    pltpu.async_copy(rv[b].at[pl.ds(j * SUB, SUB)],
                                     acc.at[edv[b].at[1, j]], ss[b],
                                     add=True)

        _wait_scatter(0)
        _wait_scatter(1)
        plsc.subcore_barrier()

        def _writeback(dst):
            base = s * WPT
            pltpu.sync_copy(acc.at[pl.ds(base, C)],
                            dst.at[pl.ds(p * NH + base, C)])

            @pl.when(s < NS - 1)
            def _r1():
                pltpu.sync_copy(acc.at[pl.ds(base + C, WPT - C)],
                                dst.at[pl.ds(p * NH + base + C, WPT - C)])

            @pl.when(s == NS - 1)
            def _r2():
                pltpu.sync_copy(acc.at[pl.ds(base + C, WPT_LAST - C)],
                                dst.at[pl.ds(p * NH + base + C, WPT_LAST - C)])

        @pl.when(c == 0)
        def _w0():
            _writeback(o0)

        @pl.when(c == 1)
        def _w1():
            _writeback(o1)


_layer = pl.kernel(
    _layer_body,
    out_type=(
        jax.ShapeDtypeStruct((NN, H), f32),
        jax.ShapeDtypeStruct((NN, H), f32),
    ),
    mesh=_mesh,
    compiler_params=pltpu.CompilerParams(use_tc_tiling_on_sc=False, needs_layout_passes=False),
    scratch_types=[
        pltpu.VMEM((3, NSTREAM, SUB), i32),
        pltpu.VMEM((3, NSTREAM, SUB), i32),
        pltpu.VMEM((C, H), f32),
        pltpu.VMEM((C, H), f32),
        pltpu.VMEM_SHARED((ACC_ROWS, H), f32),
        pltpu.SemaphoreType.DMA,
        pltpu.SemaphoreType.DMA,
        pltpu.SemaphoreType.DMA,
        pltpu.SemaphoreType.DMA,
    ],
)

BPW = B // (NC * NS)


def _score_body(e00, e01, e10, e11, e20, e21, e30, e31, uids, iids, out,
                idx_u, idx_i, gbuf, sc_v, sem):
    c = lax.axis_index("c")
    s = lax.axis_index("s")
    w = s * NC + c

    pltpu.sync_copy(uids.at[pl.ds(w * BPW, BPW)], idx_u)
    pltpu.sync_copy(iids.at[pl.ds(w * BPW, BPW)], idx_i)

    tabs = [e00, e01, e10, e11, e20, e21, e30, e31]
    descs = []
    for t in range(8):
        descs.append(pltpu.async_copy(tabs[t].at[idx_u], gbuf.at[t], sem))
        descs.append(pltpu.async_copy(tabs[t].at[idx_i], gbuf.at[8 + t], sem))
    for d in descs:
        d.wait()

    @pl.loop(0, BPW, step=16)
    def _dot(j0):
        lane = lax.iota(i32, 16)
        tot = jnp.zeros((16,), f32)
        for jj in range(16):
            j = j0 + jj
            u0 = gbuf[0, j, :] + gbuf[2, j, :] + gbuf[4, j, :] + gbuf[6, j, :]
            u1 = gbuf[1, j, :] + gbuf[3, j, :] + gbuf[5, j, :] + gbuf[7, j, :]
            i0 = gbuf[8, j, :] + gbuf[10, j, :] + gbuf[12, j, :] + gbuf[14, j, :]
            i1 = gbuf[9, j, :] + gbuf[11, j, :] + gbuf[13, j, :] + gbuf[15, j, :]
            p = u0 * i0 + u1 * i1
            ssum = jnp.sum(p) * (1.0 / 16.0)
            tot = jnp.where(lane == jj, lax.broadcast_in_dim(ssum, (16,), ()), tot)
        sc_v[pl.ds(j0, 16)] = tot

    pltpu.sync_copy(sc_v, out.at[pl.ds(w * BPW, BPW)])


_score = pl.kernel(
    _score_body,
    out_type=jax.ShapeDtypeStruct((B,), f32),
    mesh=_mesh,
    compiler_params=pltpu.CompilerParams(use_tc_tiling_on_sc=False, needs_layout_passes=False),
    scratch_types=[
        pltpu.VMEM((BPW,), i32),
        pltpu.VMEM((BPW,), i32),
        pltpu.VMEM((16, BPW, H), f32),
        pltpu.VMEM((BPW,), f32),
        pltpu.SemaphoreType.DMA,
    ],
)


def kernel(user_ids, item_ids, adj_indices, adj_values, user_emb_w, item_emb_w):
    row = adj_indices[0].astype(i32)
    col = adj_indices[1].astype(i32)
    pad = EPAD - E
    colp = jnp.concatenate([col, jnp.zeros((pad,), i32)])
    rowp = jnp.concatenate([row, jnp.zeros((pad,), i32)])
    valp = jnp.concatenate([adj_values.astype(f32), jnp.zeros((pad,), f32)])
    col3 = colp.reshape(NS, CHUNKS, NSTREAM, SUB)
    row3 = rowp.reshape(NS, CHUNKS, NSTREAM, SUB)
    val3 = lax.bitcast_convert_type(valp, i32).reshape(NS, CHUNKS, NSTREAM, SUB)
    edata = jnp.stack([col3, row3, val3], axis=2)

    a0 = jnp.concatenate([user_emb_w[:, :H], item_emb_w[:, :H]], axis=0)
    a1 = jnp.concatenate([user_emb_w[:, H:], item_emb_w[:, H:]], axis=0)
    embs = [(a0, a1)]
    for _ in range(NLAYERS):
        a0, a1 = _layer(a0, a1, edata)
        embs.append((a0, a1))

    uids = user_ids.astype(i32)
    iids = item_ids.astype(i32) + NU
    return _score(embs[0][0], embs[0][1], embs[1][0], embs[1][1],
                  embs[2][0], embs[2][1], embs[3][0], embs[3][1],
                  uids, iids)
